# baseline (speedup 1.0000x reference)
import jax
import jax.numpy as jnp
from jax.experimental import pallas as pl
from jax.experimental.pallas import tpu as pltpu

_BLOCK_ROWS = 512
_EPS = 1e-8


def _fused_kernel(x_ref, amt_ref, out_ref):
    x = x_ref[...]
    amt = amt_ref[...]
    dots = jnp.dot(x, amt, preferred_element_type=jnp.float32)
    xn = jnp.sqrt(jnp.sum(x * x, axis=1, keepdims=True))
    yn = jnp.sqrt(jnp.sum(amt * amt, axis=0, keepdims=True))
    den = jnp.maximum(xn * yn, _EPS)
    sims = dots / den
    s0 = sims[:, 0:1]
    s1 = sims[:, 1:2]
    out_ref[...] = (s1 > s0).astype(jnp.int32)


def kernel(enc_hvs, am):
    n, d = enc_hvs.shape
    amt = am.astype(jnp.float32).T
    grid = n // _BLOCK_ROWS
    out = pl.pallas_call(
        _fused_kernel,
        grid=(grid,),
        in_specs=[
            pl.BlockSpec((_BLOCK_ROWS, d), lambda i: (i, 0)),
            pl.BlockSpec((d, 2), lambda i: (0, 0)),
        ],
        out_specs=pl.BlockSpec((_BLOCK_ROWS, 1), lambda i: (i, 0)),
        out_shape=jax.ShapeDtypeStruct((n, 1), jnp.int32),
        compiler_params=pltpu.CompilerParams(
            dimension_semantics=("arbitrary",),
        ),
    )(enc_hvs, amt)
    return out.reshape(n)


# speedup vs baseline: 1.0012x; 1.0012x over previous
import jax
import jax.numpy as jnp
from jax.experimental import pallas as pl
from jax.experimental.pallas import tpu as pltpu

_BLOCK_ROWS = 128
_NSTREAMS = 4
_EPS = 1e-8


def _fused_kernel(*refs):
    x_refs = refs[:_NSTREAMS]
    amt_ref = refs[_NSTREAMS]
    out_ref = refs[_NSTREAMS + 1]
    amt = amt_ref[...]
    yn = jnp.sqrt(jnp.sum(amt * amt, axis=0, keepdims=True))
    for j in range(_NSTREAMS):
        x = x_refs[j][...]
        dots = jnp.dot(x, amt, preferred_element_type=jnp.float32)
        xn = jnp.sqrt(jnp.sum(x * x, axis=1, keepdims=True))
        den = jnp.maximum(xn * yn, _EPS)
        sims = dots / den
        s0 = sims[:, 0:1]
        s1 = sims[:, 1:2]
        out_ref[j * _BLOCK_ROWS:(j + 1) * _BLOCK_ROWS, :] = (
            (s1 > s0).astype(jnp.int32))


def kernel(enc_hvs, am):
    n, d = enc_hvs.shape
    amt = am.astype(jnp.float32).T
    rows_per_step = _BLOCK_ROWS * _NSTREAMS
    grid = n // rows_per_step

    def make_spec(j):
        return pl.BlockSpec((_BLOCK_ROWS, d),
                            lambda i, j=j: (i * _NSTREAMS + j, 0))

    out = pl.pallas_call(
        _fused_kernel,
        grid=(grid,),
        in_specs=[make_spec(j) for j in range(_NSTREAMS)]
        + [pl.BlockSpec((d, 2), lambda i: (0, 0))],
        out_specs=pl.BlockSpec((rows_per_step, 1), lambda i: (i, 0)),
        out_shape=jax.ShapeDtypeStruct((n, 1), jnp.int32),
        compiler_params=pltpu.CompilerParams(
            dimension_semantics=("arbitrary",),
        ),
    )(*([enc_hvs] * _NSTREAMS), amt)
    return out.reshape(n)


# speedup vs baseline: 1.0038x; 1.0025x over previous
import jax
import jax.numpy as jnp
from jax.experimental import pallas as pl
from jax.experimental.pallas import tpu as pltpu

_BLOCK_ROWS = 128
_NSTREAMS = 4
_EPS = 1e-8


def _fused_kernel(*refs):
    x_refs = refs[:_NSTREAMS]
    amt_ref = refs[_NSTREAMS]
    out_ref = refs[_NSTREAMS + 1]
    amt = amt_ref[...]
    yn = jnp.sqrt(jnp.sum(amt * amt, axis=0, keepdims=True))
    for j in range(_NSTREAMS):
        x = x_refs[j][...]
        dots = jnp.dot(x, amt, preferred_element_type=jnp.float32)
        xn = jnp.sqrt(jnp.sum(x * x, axis=1, keepdims=True))
        den = jnp.maximum(xn * yn, _EPS)
        sims = dots / den
        s0 = sims[:, 0:1]
        s1 = sims[:, 1:2]
        out_ref[j * _BLOCK_ROWS:(j + 1) * _BLOCK_ROWS, :] = (
            (s1 > s0).astype(jnp.int32))


def kernel(enc_hvs, am):
    n, d = enc_hvs.shape
    amt = am.astype(jnp.float32).T
    rows_per_step = _BLOCK_ROWS * _NSTREAMS
    grid = n // rows_per_step

    def make_spec(j):
        return pl.BlockSpec((_BLOCK_ROWS, d),
                            lambda i, j=j: (i * _NSTREAMS + j, 0))

    out = pl.pallas_call(
        _fused_kernel,
        grid=(grid,),
        in_specs=[make_spec(j) for j in range(_NSTREAMS)]
        + [pl.BlockSpec((d, 2), lambda i: (0, 0))],
        out_specs=pl.BlockSpec((rows_per_step, 1), lambda i: (i, 0)),
        out_shape=jax.ShapeDtypeStruct((n, 1), jnp.int32),
        compiler_params=pltpu.CompilerParams(
            dimension_semantics=("parallel",),
        ),
    )(*([enc_hvs] * _NSTREAMS), amt)
    return out.reshape(n)
